# baseline (speedup 1.0000x reference)
import functools

import jax
import jax.numpy as jnp
from jax import lax
from jax.experimental import pallas as pl
from jax.experimental.pallas import tpu as pltpu
from jax.experimental.pallas import tpu_sc as plsc

N = 10000
E = 160000
D = 256
DH = 128
NS = 16
NC = 2
B = 96
NBLK = 112
HBLK = 24
EPT = NBLK * B
EPAD = NS * EPT
PHASES = ((0, 24), (24, 24), (48, 24), (72, 24), (96, 16))
RPT = 632
NPAD = NS * RPT
ZR = 640

_CHUNKS = tuple((k * B, B) for k in range(6)) + ((6 * B, RPT - 6 * B),)


def _pre_body(x_ref, wl_ref, wr_ref, b_ref, ya_ref, yb_ref, s_ref):
    x = x_ref[...]
    y = jnp.dot(x, wl_ref[...], preferred_element_type=jnp.float32)
    ya_ref[...] = y[:, :DH]
    yb_ref[...] = y[:, DH:]
    s_ref[...] = jnp.dot(x, wr_ref[...], preferred_element_type=jnp.float32) + b_ref[...]


def _pre(x, wl, wr, b):
    bn = 2000
    return pl.pallas_call(
        _pre_body,
        grid=(N // bn,),
        in_specs=[
            pl.BlockSpec((bn, D), lambda i: (i, 0)),
            pl.BlockSpec((D, D), lambda i: (0, 0)),
            pl.BlockSpec((D, D), lambda i: (0, 0)),
            pl.BlockSpec((1, D), lambda i: (0, 0)),
        ],
        out_specs=[
            pl.BlockSpec((bn, DH), lambda i: (i, 0)),
            pl.BlockSpec((bn, DH), lambda i: (i, 0)),
            pl.BlockSpec((bn, D), lambda i: (i, 0)),
        ],
        out_shape=[
            jax.ShapeDtypeStruct((N, DH), jnp.float32),
            jax.ShapeDtypeStruct((N, DH), jnp.float32),
            jax.ShapeDtypeStruct((N, D), jnp.float32),
        ],
    )(x, wl, wr, b)


def _finish(a_ref, b_ref, d0_ref, d1_ref, s_ref, relu):
    inv = 1.0 / jnp.maximum(d0_ref[:, 0:1] + d1_ref[:, 0:1], 1.0)
    o = jnp.concatenate([a_ref[...], b_ref[...]], axis=1) * inv + s_ref[...]
    if relu:
        o = jnp.maximum(o, 0.0)
    return o


def _mid_body(a_ref, b_ref, d0_ref, d1_ref, s_ref, wl_ref, wr_ref, bb_ref,
              ya_ref, yb_ref, s2_ref):
    h = _finish(a_ref, b_ref, d0_ref, d1_ref, s_ref, relu=True)
    y = jnp.dot(h, wl_ref[...], preferred_element_type=jnp.float32)
    ya_ref[...] = y[:, :DH]
    yb_ref[...] = y[:, DH:]
    s2_ref[...] = jnp.dot(h, wr_ref[...], preferred_element_type=jnp.float32) + bb_ref[...]


def _mid(acc_a, acc_b, d0, d1, s, wl, wr, b):
    bn = 1000
    return pl.pallas_call(
        _mid_body,
        grid=(N // bn,),
        in_specs=[
            pl.BlockSpec((bn, DH), lambda i: (i, 0)),
            pl.BlockSpec((bn, DH), lambda i: (i, 0)),
            pl.BlockSpec((bn, DH), lambda i: (i, 0)),
            pl.BlockSpec((bn, DH), lambda i: (i, 0)),
            pl.BlockSpec((bn, D), lambda i: (i, 0)),
            pl.BlockSpec((D, D), lambda i: (0, 0)),
            pl.BlockSpec((D, D), lambda i: (0, 0)),
            pl.BlockSpec((1, D), lambda i: (0, 0)),
        ],
        out_specs=[
            pl.BlockSpec((bn, DH), lambda i: (i, 0)),
            pl.BlockSpec((bn, DH), lambda i: (i, 0)),
            pl.BlockSpec((bn, D), lambda i: (i, 0)),
        ],
        out_shape=[
            jax.ShapeDtypeStruct((N, DH), jnp.float32),
            jax.ShapeDtypeStruct((N, DH), jnp.float32),
            jax.ShapeDtypeStruct((N, D), jnp.float32),
        ],
    )(acc_a, acc_b, d0, d1, s, wl, wr, b)


def _post_body(a_ref, b_ref, d0_ref, d1_ref, s_ref, o_ref, *, relu=False):
    o_ref[...] = _finish(a_ref, b_ref, d0_ref, d1_ref, s_ref, relu=relu)


def _post(acc_a, acc_b, d0, d1, s, relu=False):
    bn = 2000
    return pl.pallas_call(
        functools.partial(_post_body, relu=relu),
        grid=(N // bn,),
        in_specs=[
            pl.BlockSpec((bn, DH), lambda i: (i, 0)),
            pl.BlockSpec((bn, DH), lambda i: (i, 0)),
            pl.BlockSpec((bn, DH), lambda i: (i, 0)),
            pl.BlockSpec((bn, DH), lambda i: (i, 0)),
            pl.BlockSpec((bn, D), lambda i: (i, 0)),
        ],
        out_specs=pl.BlockSpec((bn, D), lambda i: (i, 0)),
        out_shape=jax.ShapeDtypeStruct((N, D), jnp.float32),
    )(acc_a, acc_b, d0, d1, s)


def _make_mesh():
    return plsc.VectorSubcoreMesh(core_axis_name="c", subcore_axis_name="s",
                                  num_cores=NC, num_subcores=NS)


def _sc_run(table, acc_out, src3, dst3, zeros, acc_sh, src_v, dst_v,
            rows0, rows1, sem0, sem1, s):
    r0 = s * RPT
    pltpu.sync_copy(zeros.at[pl.ds(0, B)], rows0)
    for off, ln in _CHUNKS:
        pltpu.sync_copy(rows0.at[pl.ds(0, ln)],
                        acc_sh.at[pl.ds(r0 + off, ln)])
    plsc.subcore_barrier()

    def pair(p, carry):
        j0 = 2 * p
        j1 = j0 + 1
        pltpu.async_copy(table.at[src_v.at[j0]], rows0, sem0).wait()
        g1 = pltpu.async_copy(table.at[src_v.at[j1]], rows1, sem1)
        pltpu.sync_copy(rows0, acc_sh.at[dst_v.at[j0]], add=True)
        g1.wait()
        pltpu.sync_copy(rows1, acc_sh.at[dst_v.at[j1]], add=True)
        return carry

    for off, nb in PHASES:
        pltpu.sync_copy(src3.at[s, pl.ds(off, nb)],
                        src_v.at[pl.ds(0, nb)])
        pltpu.sync_copy(dst3.at[s, pl.ds(off, nb)],
                        dst_v.at[pl.ds(0, nb)])
        lax.fori_loop(0, nb // 2, pair, 0)
    plsc.subcore_barrier()
    for off, ln in _CHUNKS:
        pltpu.sync_copy(acc_sh.at[pl.ds(r0 + off, ln)],
                        rows0.at[pl.ds(0, ln)])
        pltpu.sync_copy(rows0.at[pl.ds(0, ln)],
                        acc_out.at[pl.ds(r0 + off, ln)])


def _make_sc_agg():
    @functools.partial(
        pl.kernel, mesh=_make_mesh(),
        out_type=[
            jax.ShapeDtypeStruct((NPAD, DH), jnp.float32),
            jax.ShapeDtypeStruct((NPAD, DH), jnp.float32),
        ],
        scratch_types=[
            pltpu.VMEM((HBLK, B), jnp.int32),
            pltpu.VMEM((HBLK, B), jnp.int32),
            pltpu.VMEM((B, DH), jnp.float32),
            pltpu.VMEM((B, DH), jnp.float32),
            pltpu.VMEM_SHARED((NPAD, DH), jnp.float32),
            pltpu.SemaphoreType.DMA,
            pltpu.SemaphoreType.DMA,
        ])
    def k(ya, yb, src3, dst3, zeros,
          acc_a_o, acc_b_o,
          src_v, dst_v, rows0, rows1, acc_sh, sem0, sem1):
        c = lax.axis_index("c")
        s = lax.axis_index("s")

        @pl.when(c == 0)
        def _():
            _sc_run(ya, acc_a_o, src3, dst3, zeros, acc_sh, src_v,
                    dst_v, rows0, rows1, sem0, sem1, s)

        @pl.when(c == 1)
        def _():
            _sc_run(yb, acc_b_o, src3, dst3, zeros, acc_sh, src_v,
                    dst_v, rows0, rows1, sem0, sem1, s)
    return k


def _make_sc_deg():
    HB2 = NBLK // 2

    @functools.partial(
        pl.kernel, mesh=_make_mesh(),
        out_type=[
            jax.ShapeDtypeStruct((NPAD, DH), jnp.float32),
            jax.ShapeDtypeStruct((NPAD, DH), jnp.float32),
        ],
        scratch_types=[
            pltpu.VMEM((HB2, B), jnp.int32),
            pltpu.VMEM((B, DH), jnp.float32),
            pltpu.VMEM_SHARED((NPAD, DH), jnp.float32),
        ])
    def k(dst3, ones128, zeros, deg0_o, deg1_o, dst_v, ones_v, deg_sh):
        c = lax.axis_index("c")
        s = lax.axis_index("s")
        r0 = s * RPT

        pltpu.sync_copy(zeros.at[pl.ds(0, B)], ones_v)
        for off, ln in _CHUNKS:
            pltpu.sync_copy(ones_v.at[pl.ds(0, ln)],
                            deg_sh.at[pl.ds(r0 + off, ln)])
        pltpu.sync_copy(ones128, ones_v)
        plsc.subcore_barrier()

        def body(j, carry):
            pltpu.sync_copy(ones_v, deg_sh.at[dst_v.at[j]], add=True)
            return carry

        def run(ph, nb, deg_o):
            pltpu.sync_copy(dst3.at[s, pl.ds(ph * HB2, nb)],
                            dst_v.at[pl.ds(0, nb)])
            lax.fori_loop(0, nb, body, 0)
            plsc.subcore_barrier()
            pltpu.sync_copy(deg_sh.at[pl.ds(r0, RPT)],
                            deg_o.at[pl.ds(r0, RPT)])

        @pl.when(c == 0)
        def _():
            run(0, HB2, deg0_o)

        @pl.when(c == 1)
        def _():
            run(1, HB2, deg1_o)
    return k


_sc_agg = _make_sc_agg()
_sc_deg = _make_sc_deg()


def kernel(x, edge_index, W_l1, W_r1, b1, W_l2, W_r2, b2):
    npad_e = EPAD - E
    src_p = jnp.concatenate(
        [edge_index[0], jnp.zeros((npad_e,), jnp.int32)])
    dst_p = jnp.concatenate(
        [edge_index[1], jnp.full((npad_e,), N, jnp.int32)])
    src3 = src_p.reshape(NS, NBLK, B)
    dst3 = dst_p.reshape(NS, NBLK, B)
    zeros = jnp.zeros((ZR, DH), jnp.float32)
    ones128 = jnp.ones((B, DH), jnp.float32)
    b1r = b1.reshape(1, D)
    b2r = b2.reshape(1, D)

    d0, d1 = _sc_deg(dst3, ones128, zeros)
    ya1, yb1, s1 = _pre(x, W_l1, W_r1, b1r)
    acc_a1, acc_b1 = _sc_agg(ya1, yb1, src3, dst3, zeros)
    h = _post(acc_a1, acc_b1, d0, d1, s1, relu=True)
    ya2, yb2, s2 = _pre(h, W_l2, W_r2, b2r)
    acc_a2, acc_b2 = _sc_agg(ya2, yb2, src3, dst3, zeros)
    return _post(acc_a2, acc_b2, d0, d1, s2)


# speedup vs baseline: 2.4635x; 2.4635x over previous
import functools

import jax
import jax.numpy as jnp
from jax import lax
from jax.experimental import pallas as pl
from jax.experimental.pallas import tpu as pltpu
from jax.experimental.pallas import tpu_sc as plsc

N = 10000
E = 160000
D = 256
DH = 128
NS = 16
NC = 2
B = 128
NBLK = 79
HBLK = 40
EPT = NBLK * B
EPAD = NS * EPT
RPT = 632
NPAD = NS * RPT
ZR = 640

_CHUNKS = tuple((k * B, B) for k in range(4)) + ((4 * B, RPT - 4 * B),)


def _pre_body(x_ref, wl_ref, wr_ref, b_ref, ya_ref, yb_ref, s_ref):
    x = x_ref[...]
    y = jnp.dot(x, wl_ref[...], preferred_element_type=jnp.float32)
    ya_ref[...] = y[:, :DH]
    yb_ref[...] = y[:, DH:]
    s_ref[...] = jnp.dot(x, wr_ref[...], preferred_element_type=jnp.float32) + b_ref[...]


def _pre(x, wl, wr, b):
    bn = 1000
    return pl.pallas_call(
        _pre_body,
        grid=(N // bn,),
        in_specs=[
            pl.BlockSpec((bn, D), lambda i: (i, 0)),
            pl.BlockSpec((D, D), lambda i: (0, 0)),
            pl.BlockSpec((D, D), lambda i: (0, 0)),
            pl.BlockSpec((1, D), lambda i: (0, 0)),
        ],
        out_specs=[
            pl.BlockSpec((bn, DH), lambda i: (i, 0)),
            pl.BlockSpec((bn, DH), lambda i: (i, 0)),
            pl.BlockSpec((bn, D), lambda i: (i, 0)),
        ],
        out_shape=[
            jax.ShapeDtypeStruct((N, DH), jnp.float32),
            jax.ShapeDtypeStruct((N, DH), jnp.float32),
            jax.ShapeDtypeStruct((N, D), jnp.float32),
        ],
    )(x, wl, wr, b)


def _finish(a_ref, b_ref, d0_ref, d1_ref, s_ref, relu):
    inv = 1.0 / jnp.maximum(d0_ref[:, 0:1] + d1_ref[:, 0:1], 1.0)
    o = jnp.concatenate([a_ref[...], b_ref[...]], axis=1) * inv + s_ref[...]
    if relu:
        o = jnp.maximum(o, 0.0)
    return o


def _post_body(a_ref, b_ref, d0_ref, d1_ref, s_ref, o_ref, *, relu=False):
    o_ref[...] = _finish(a_ref, b_ref, d0_ref, d1_ref, s_ref, relu=relu)


def _post(acc_a, acc_b, d0, d1, s, relu=False):
    bn = 1000
    return pl.pallas_call(
        functools.partial(_post_body, relu=relu),
        grid=(N // bn,),
        in_specs=[
            pl.BlockSpec((bn, DH), lambda i: (i, 0)),
            pl.BlockSpec((bn, DH), lambda i: (i, 0)),
            pl.BlockSpec((bn, DH), lambda i: (i, 0)),
            pl.BlockSpec((bn, DH), lambda i: (i, 0)),
            pl.BlockSpec((bn, D), lambda i: (i, 0)),
        ],
        out_specs=pl.BlockSpec((bn, D), lambda i: (i, 0)),
        out_shape=jax.ShapeDtypeStruct((N, D), jnp.float32),
    )(acc_a, acc_b, d0, d1, s)


def _make_mesh():
    return plsc.VectorSubcoreMesh(core_axis_name="c", subcore_axis_name="s",
                                  num_cores=NC, num_subcores=NS)


def _sc_run(table, acc_out, src3, dst3, zeros, acc_sh, src_v, dst_v,
            rows0, sem0, s):
    r0 = s * RPT
    bufs = (rows0,)
    sems = (sem0,)
    pltpu.sync_copy(zeros.at[pl.ds(0, B)], rows0)
    for off, ln in _CHUNKS:
        pltpu.sync_copy(rows0.at[pl.ds(0, ln)],
                        acc_sh.at[pl.ds(r0 + off, ln)])
    plsc.subcore_barrier()

    def body(j, carry):
        pltpu.async_copy(table.at[src_v.at[j]], bufs[0], sems[0]).wait()
        pltpu.sync_copy(bufs[0], acc_sh.at[dst_v.at[j]], add=True)
        return carry

    for ph, nb in ((0, HBLK), (1, NBLK - HBLK)):
        pltpu.sync_copy(src3.at[s, pl.ds(ph * HBLK, nb)],
                        src_v.at[pl.ds(0, nb)])
        pltpu.sync_copy(dst3.at[s, pl.ds(ph * HBLK, nb)],
                        dst_v.at[pl.ds(0, nb)])
        lax.fori_loop(0, nb, body, 0)
    plsc.subcore_barrier()
    for off, ln in _CHUNKS:
        pltpu.sync_copy(acc_sh.at[pl.ds(r0 + off, ln)],
                        rows0.at[pl.ds(0, ln)])
        pltpu.sync_copy(rows0.at[pl.ds(0, ln)],
                        acc_out.at[pl.ds(r0 + off, ln)])


def _make_sc_agg():
    @functools.partial(
        pl.kernel, mesh=_make_mesh(),
        out_type=[
            jax.ShapeDtypeStruct((NPAD, DH), jnp.float32),
            jax.ShapeDtypeStruct((NPAD, DH), jnp.float32),
        ],
        scratch_types=[
            pltpu.VMEM((HBLK, B), jnp.int32),
            pltpu.VMEM((HBLK, B), jnp.int32),
            pltpu.VMEM((B, DH), jnp.float32),
            pltpu.VMEM_SHARED((NPAD, DH), jnp.float32),
            pltpu.SemaphoreType.DMA,
        ])
    def k(ya, yb, src3, dst3, zeros,
          acc_a_o, acc_b_o,
          src_v, dst_v, rows0, acc_sh, sem0):
        c = lax.axis_index("c")
        s = lax.axis_index("s")

        @pl.when(c == 0)
        def _():
            _sc_run(ya, acc_a_o, src3, dst3, zeros, acc_sh, src_v,
                    dst_v, rows0, sem0, s)

        @pl.when(c == 1)
        def _():
            _sc_run(yb, acc_b_o, src3, dst3, zeros, acc_sh, src_v,
                    dst_v, rows0, sem0, s)
    return k


def _make_sc_deg():
    HB2 = HBLK

    @functools.partial(
        pl.kernel, mesh=_make_mesh(),
        out_type=[
            jax.ShapeDtypeStruct((NPAD, DH), jnp.float32),
            jax.ShapeDtypeStruct((NPAD, DH), jnp.float32),
        ],
        scratch_types=[
            pltpu.VMEM((HB2, B), jnp.int32),
            pltpu.VMEM((B, DH), jnp.float32),
            pltpu.VMEM_SHARED((NPAD, DH), jnp.float32),
        ])
    def k(dst3, ones128, zeros, deg0_o, deg1_o, dst_v, ones_v, deg_sh):
        c = lax.axis_index("c")
        s = lax.axis_index("s")
        r0 = s * RPT

        pltpu.sync_copy(zeros.at[pl.ds(0, B)], ones_v)
        for off, ln in _CHUNKS:
            pltpu.sync_copy(ones_v.at[pl.ds(0, ln)],
                            deg_sh.at[pl.ds(r0 + off, ln)])
        pltpu.sync_copy(ones128, ones_v)
        plsc.subcore_barrier()

        def body(j, carry):
            pltpu.sync_copy(ones_v, deg_sh.at[dst_v.at[j]], add=True)
            return carry

        def run(ph, nb, deg_o):
            pltpu.sync_copy(dst3.at[s, pl.ds(ph * HB2, nb)],
                            dst_v.at[pl.ds(0, nb)])
            lax.fori_loop(0, nb, body, 0)
            plsc.subcore_barrier()
            pltpu.sync_copy(deg_sh.at[pl.ds(r0, RPT)],
                            deg_o.at[pl.ds(r0, RPT)])

        @pl.when(c == 0)
        def _():
            run(0, HBLK, deg0_o)

        @pl.when(c == 1)
        def _():
            run(1, NBLK - HBLK, deg1_o)
    return k


_sc_agg = _make_sc_agg()
_sc_deg = _make_sc_deg()


def kernel(x, edge_index, W_l1, W_r1, b1, W_l2, W_r2, b2):
    npad_e = EPAD - E
    src_p = jnp.concatenate(
        [edge_index[0], jnp.zeros((npad_e,), jnp.int32)])
    dst_p = jnp.concatenate(
        [edge_index[1], jnp.full((npad_e,), N, jnp.int32)])
    src3 = src_p.reshape(NS, NBLK, B)
    dst3 = dst_p.reshape(NS, NBLK, B)
    zeros = jnp.zeros((ZR, DH), jnp.float32)
    ones128 = jnp.ones((B, DH), jnp.float32)
    b1r = b1.reshape(1, D)
    b2r = b2.reshape(1, D)

    d0, d1 = _sc_deg(dst3, ones128, zeros)
    ya1, yb1, s1 = _pre(x, W_l1, W_r1, b1r)
    acc_a1, acc_b1 = _sc_agg(ya1, yb1, src3, dst3, zeros)
    h = _post(acc_a1, acc_b1, d0, d1, s1, relu=True)
    ya2, yb2, s2 = _pre(h, W_l2, W_r2, b2r)
    acc_a2, acc_b2 = _sc_agg(ya2, yb2, src3, dst3, zeros)
    return _post(acc_a2, acc_b2, d0, d1, s2)
